# baseline (speedup 1.0000x reference)
import functools

import jax
import jax.numpy as jnp
from jax import lax
from jax.experimental import pallas as pl
from jax.experimental.pallas import tpu as pltpu
from jax.experimental.pallas import tpu_sc as plsc

_CHUNK = 128


def _sc_gather(table, idx):
    info = plsc.get_sparse_core_info()
    nc, ns = info.num_cores, info.num_subcores
    nw = nc * ns
    n = idx.shape[0]
    d = table.shape[1]
    per_w = n // nw
    nchunk = per_w // _CHUNK
    idx3 = idx.reshape(nw, nchunk, _CHUNK)
    mesh = plsc.VectorSubcoreMesh(core_axis_name="c", subcore_axis_name="s")

    @functools.partial(
        pl.kernel,
        mesh=mesh,
        out_type=jax.ShapeDtypeStruct((n, d), jnp.float32),
        scratch_types=[
            pltpu.VMEM((nchunk, _CHUNK), jnp.int32),
            pltpu.VMEM((per_w, d), jnp.float32),
            pltpu.SemaphoreType.DMA,
        ],
        compiler_params=pltpu.CompilerParams(use_tc_tiling_on_sc=False),
    )
    def k(table_hbm, idx_hbm, out_hbm, idx_v, rows_v, sem):
        wid = lax.axis_index("s") * nc + lax.axis_index("c")
        base = wid * per_w
        pltpu.sync_copy(idx_hbm.at[wid], idx_v)
        copies = [
            pltpu.async_copy(
                table_hbm.at[idx_v.at[j]],
                rows_v.at[pl.ds(j * _CHUNK, _CHUNK)],
                sem,
            )
            for j in range(nchunk)
        ]
        for c in copies:
            c.wait()
        pltpu.sync_copy(rows_v, out_hbm.at[pl.ds(base, per_w)])

    return k(table, idx3)


def _tc_ht(x, w1, b1c):
    bsz, ld = x.shape
    d = w1.shape[1]

    def body(x_ref, w1_ref, b1_ref, out_ref):
        ht = (
            lax.dot_general(
                w1_ref[...], x_ref[...],
                (((0,), (1,)), ((), ())),
                preferred_element_type=jnp.float32,
            )
            + b1_ref[...]
        )
        out_ref[...] = jnp.concatenate(
            [ht, jnp.ones((1, bsz), jnp.float32)], axis=0
        )

    return pl.pallas_call(
        body,
        out_shape=jax.ShapeDtypeStruct((d + 1, bsz), jnp.float32),
    )(x, w1, b1c)


def _tc_out_t(ht_aug, w2, b2r, tv):
    da, bsz = ht_aug.shape
    d = da - 1
    v = w2.shape[1]
    ntiles = pl.cdiv(v, tv)

    def body(ht_ref, w2_ref, b2_ref, out_ref):
        w2aug = jnp.concatenate([w2_ref[...], b2_ref[...]], axis=0)
        out_ref[...] = lax.dot_general(
            w2aug.astype(jnp.bfloat16),
            ht_ref[...].astype(jnp.bfloat16),
            (((0,), (0,)), ((), ())),
            preferred_element_type=jnp.float32,
        )

    return pl.pallas_call(
        body,
        grid=(ntiles,),
        in_specs=[
            pl.BlockSpec((da, bsz), lambda i: (0, 0)),
            pl.BlockSpec((d, tv), lambda i: (0, i)),
            pl.BlockSpec((1, tv), lambda i: (0, i)),
        ],
        out_specs=pl.BlockSpec((tv, bsz), lambda i: (i, 0)),
        out_shape=jax.ShapeDtypeStruct((v, bsz), jnp.float32),
        compiler_params=pltpu.CompilerParams(
            dimension_semantics=("parallel",),
        ),
    )(ht_aug, w2, b2r)


def kernel(indices, emb_table, W1, b1, W2, b2):
    bsz, seq = indices.shape
    d = emb_table.shape[1]
    idx_flat = indices.reshape(-1).astype(jnp.int32)
    rows = _sc_gather(emb_table, idx_flat)
    x = rows.reshape(bsz, seq * d)
    ht = _tc_ht(x, W1, b1.reshape(d, 1))
    out_t = _tc_out_t(ht, W2, b2.reshape(1, -1), tv=4096)
    return out_t.T


# speedup vs baseline: 1.0074x; 1.0074x over previous
import functools

import jax
import jax.numpy as jnp
from jax import lax
from jax.experimental import pallas as pl
from jax.experimental.pallas import tpu as pltpu
from jax.experimental.pallas import tpu_sc as plsc

_CHUNK = 128


def _sc_gather(table, idx):
    info = plsc.get_sparse_core_info()
    nc, ns = info.num_cores, info.num_subcores
    nw = nc * ns
    n = idx.shape[0]
    d = table.shape[1]
    per_w = n // nw
    nchunk = per_w // _CHUNK
    idx3 = idx.reshape(nw, nchunk, _CHUNK)
    mesh = plsc.VectorSubcoreMesh(core_axis_name="c", subcore_axis_name="s")

    @functools.partial(
        pl.kernel,
        mesh=mesh,
        out_type=jax.ShapeDtypeStruct((n, d), jnp.float32),
        scratch_types=[
            pltpu.VMEM((nchunk, _CHUNK), jnp.int32),
            pltpu.VMEM((per_w, d), jnp.float32),
            pltpu.SemaphoreType.DMA,
        ],
        compiler_params=pltpu.CompilerParams(use_tc_tiling_on_sc=False),
    )
    def k(table_hbm, idx_hbm, out_hbm, idx_v, rows_v, sem):
        wid = lax.axis_index("s") * nc + lax.axis_index("c")
        base = wid * per_w
        pltpu.sync_copy(idx_hbm.at[wid], idx_v)
        copies = [
            pltpu.async_copy(
                table_hbm.at[idx_v.at[j]],
                rows_v.at[pl.ds(j * _CHUNK, _CHUNK)],
                sem,
            )
            for j in range(nchunk)
        ]
        for c in copies:
            c.wait()
        pltpu.sync_copy(rows_v, out_hbm.at[pl.ds(base, per_w)])

    return k(table, idx3)


def _tc_mlp_t(x, w1, b1c, w2, b2r, tv):
    bsz, ld = x.shape
    d = w1.shape[1]
    v = w2.shape[1]
    ntiles = pl.cdiv(v, tv)

    def body(x_ref, w1_ref, b1_ref, w2_ref, b2_ref, out_ref, ht_ref):
        @pl.when(pl.program_id(0) == 0)
        def _():
            ht = (
                lax.dot_general(
                    w1_ref[...], x_ref[...],
                    (((0,), (1,)), ((), ())),
                    preferred_element_type=jnp.float32,
                )
                + b1_ref[...]
            )
            ht_ref[...] = jnp.concatenate(
                [ht, jnp.ones((1, bsz), jnp.float32)], axis=0
            ).astype(jnp.bfloat16)

        w2aug = jnp.concatenate([w2_ref[...], b2_ref[...]], axis=0)
        out_ref[...] = lax.dot_general(
            w2aug.astype(jnp.bfloat16),
            ht_ref[...],
            (((0,), (0,)), ((), ())),
            preferred_element_type=jnp.float32,
        )

    return pl.pallas_call(
        body,
        grid=(ntiles,),
        in_specs=[
            pl.BlockSpec((bsz, ld), lambda i: (0, 0)),
            pl.BlockSpec((ld, d), lambda i: (0, 0)),
            pl.BlockSpec((d, 1), lambda i: (0, 0)),
            pl.BlockSpec((d, tv), lambda i: (0, i)),
            pl.BlockSpec((1, tv), lambda i: (0, i)),
        ],
        out_specs=pl.BlockSpec((tv, bsz), lambda i: (i, 0)),
        out_shape=jax.ShapeDtypeStruct((v, bsz), jnp.float32),
        scratch_shapes=[pltpu.VMEM((d + 1, bsz), jnp.bfloat16)],
        compiler_params=pltpu.CompilerParams(
            dimension_semantics=("arbitrary",),
        ),
    )(x, w1, b1c, w2, b2r)


def kernel(indices, emb_table, W1, b1, W2, b2):
    bsz, seq = indices.shape
    d = emb_table.shape[1]
    idx_flat = indices.reshape(-1).astype(jnp.int32)
    rows = _sc_gather(emb_table, idx_flat)
    x = rows.reshape(bsz, seq * d)
    out_t = _tc_mlp_t(
        x, W1, b1.reshape(d, 1), W2, b2.reshape(1, -1), tv=4096
    )
    return out_t.T


# speedup vs baseline: 1.0097x; 1.0022x over previous
import functools

import jax
import jax.numpy as jnp
from jax import lax
from jax.experimental import pallas as pl
from jax.experimental.pallas import tpu as pltpu
from jax.experimental.pallas import tpu_sc as plsc

_CHUNK = 128


def _sc_gather(table, idx):
    info = plsc.get_sparse_core_info()
    nc, ns = info.num_cores, info.num_subcores
    nw = nc * ns
    n = idx.shape[0]
    d = table.shape[1]
    per_w = n // nw
    nchunk = per_w // _CHUNK
    idx3 = idx.reshape(nw, nchunk, _CHUNK)
    mesh = plsc.VectorSubcoreMesh(core_axis_name="c", subcore_axis_name="s")

    @functools.partial(
        pl.kernel,
        mesh=mesh,
        out_type=jax.ShapeDtypeStruct((n, d), jnp.float32),
        scratch_types=[
            pltpu.VMEM((nchunk, _CHUNK), jnp.int32),
            pltpu.VMEM((per_w, d), jnp.float32),
            pltpu.SemaphoreType.DMA,
        ],
        compiler_params=pltpu.CompilerParams(use_tc_tiling_on_sc=False),
    )
    def k(table_hbm, idx_hbm, out_hbm, idx_v, rows_v, sem):
        wid = lax.axis_index("s") * nc + lax.axis_index("c")
        base = wid * per_w
        pltpu.sync_copy(idx_hbm.at[wid], idx_v)
        copies = [
            pltpu.async_copy(
                table_hbm.at[idx_v.at[j]],
                rows_v.at[pl.ds(j * _CHUNK, _CHUNK)],
                sem,
            )
            for j in range(nchunk)
        ]
        for c in copies:
            c.wait()
        pltpu.sync_copy(rows_v, out_hbm.at[pl.ds(base, per_w)])

    return k(table, idx3)


def _tc_mlp_t(x, w1, b1c, w2, b2r, tv):
    bsz, ld = x.shape
    d = w1.shape[1]
    v = w2.shape[1]
    ntiles = pl.cdiv(v, tv)

    def body(x_ref, w1_ref, b1_ref, w2_ref, b2_ref, out_ref, ht_ref):
        @pl.when(pl.program_id(0) == 0)
        def _():
            ht = (
                lax.dot_general(
                    w1_ref[...], x_ref[...],
                    (((0,), (1,)), ((), ())),
                    preferred_element_type=jnp.float32,
                )
                + b1_ref[...]
            )
            ht_ref[...] = jnp.concatenate(
                [ht, jnp.ones((1, bsz), jnp.float32)], axis=0
            ).astype(jnp.bfloat16)

        w2aug = jnp.concatenate([w2_ref[...], b2_ref[...]], axis=0)
        out_ref[...] = lax.dot_general(
            w2aug.astype(jnp.bfloat16),
            ht_ref[...],
            (((0,), (0,)), ((), ())),
            preferred_element_type=jnp.float32,
        )

    return pl.pallas_call(
        body,
        grid=(ntiles,),
        in_specs=[
            pl.BlockSpec((bsz, ld), lambda i: (0, 0)),
            pl.BlockSpec((ld, d), lambda i: (0, 0)),
            pl.BlockSpec((d, 1), lambda i: (0, 0)),
            pl.BlockSpec((d, tv), lambda i: (0, i)),
            pl.BlockSpec((1, tv), lambda i: (0, i)),
        ],
        out_specs=pl.BlockSpec((tv, bsz), lambda i: (i, 0)),
        out_shape=jax.ShapeDtypeStruct((v, bsz), jnp.float32),
        scratch_shapes=[pltpu.VMEM((d + 1, bsz), jnp.bfloat16)],
        compiler_params=pltpu.CompilerParams(
            dimension_semantics=("arbitrary",),
            skip_device_barrier=True,
        ),
    )(x, w1, b1c, w2, b2r)


def kernel(indices, emb_table, W1, b1, W2, b2):
    bsz, seq = indices.shape
    d = emb_table.shape[1]
    idx_flat = indices.reshape(-1).astype(jnp.int32)
    rows = _sc_gather(emb_table, idx_flat)
    x = rows.reshape(bsz, seq * d)
    out_t = _tc_mlp_t(
        x, W1, b1.reshape(d, 1), W2, b2.reshape(1, -1), tv=4096
    )
    return out_t.T
